# baseline (speedup 1.0000x reference)
import jax
import jax.numpy as jnp
from jax import lax
from jax.experimental import pallas as pl
from jax.experimental.pallas import tpu as pltpu
from jax.experimental.pallas import tpu_sc as plsc

_D = 32
_NC = 2
_NS = 16
_NW = _NC * _NS
_C = 512
_TW = 800

_SLICE_ROWS = 8192


def _lane_slice_tc(flat_padded, n):
    def body(i_ref, o_ref):
        o_ref[...] = i_ref[...].reshape(_SLICE_ROWS, 4 * _D)[:, :_D]

    return pl.pallas_call(
        body,
        grid=(n // _SLICE_ROWS,),
        in_specs=[pl.BlockSpec((_SLICE_ROWS * 4 * _D,), lambda i: (i,))],
        out_specs=pl.BlockSpec((_SLICE_ROWS, _D), lambda i: (i, 0)),
        out_shape=jax.ShapeDtypeStruct((n, _D), flat_padded.dtype),
        compiler_params=pltpu.CompilerParams(
            dimension_semantics=("parallel",)),
    )(flat_padded)


def kernel(x, table):
    b, s = x.shape
    n = b * s
    v = table.shape[0]
    idx = x.reshape(n)
    b_per_w = n // _NW
    n_chunks = b_per_w // _C
    v_per_w = v // _NW
    mesh = plsc.VectorSubcoreMesh(core_axis_name="c", subcore_axis_name="s")

    table_t = table.T

    @pl.kernel(
        out_type=jax.ShapeDtypeStruct((v, _D), table.dtype),
        mesh=mesh,
        compiler_params=pltpu.CompilerParams(
            use_tc_tiling_on_sc=False, needs_layout_passes=False),
        scratch_types=[
            pltpu.VMEM((_D, _TW), jnp.float32),
            pltpu.VMEM((_TW, _D), jnp.float32),
        ],
    )
    def transpose_kernel(tab_t_hbm, out_hbm, slab_v, outb_v):
        wid = lax.axis_index("s") * _NC + lax.axis_index("c")
        n_slabs = v // _TW
        n_rounds = (n_slabs + _NW - 1) // _NW
        e_lo = lax.iota(jnp.int32, 16)
        e_hi = e_lo + 16

        @pl.loop(0, n_rounds)
        def _(t):
            sid = wid + t * _NW

            @pl.when(sid < n_slabs)
            def _():
                c0 = sid * _TW
                pltpu.sync_copy(tab_t_hbm.at[:, pl.ds(c0, _TW)], slab_v)

                @pl.loop(0, _TW)
                def _(j):
                    jv = jnp.full((16,), j, jnp.int32)
                    outb_v[j, pl.ds(0, 16)] = plsc.load_gather(slab_v, [e_lo, jv])
                    outb_v[j, pl.ds(16, 16)] = plsc.load_gather(slab_v, [e_hi, jv])

                pltpu.sync_copy(outb_v, out_hbm.at[pl.ds(c0, _TW)])

    @pl.kernel(
        out_type=jax.ShapeDtypeStruct((n, 4 * _D), table.dtype),
        mesh=mesh,
        compiler_params=pltpu.CompilerParams(use_tc_tiling_on_sc=False),
        scratch_types=[
            pltpu.VMEM((b_per_w,), jnp.int32),
            pltpu.VMEM((_C, _D), jnp.float32),
            pltpu.VMEM((_C, _D), jnp.float32),
            pltpu.SemaphoreType.DMA,
            pltpu.SemaphoreType.DMA,
        ],
    )
    def gather_kernel(table_hbm, idx_hbm, out_hbm, idx_v, rows0, rows1, sem0, sem1):
        wid = lax.axis_index("s") * _NC + lax.axis_index("c")
        base = wid * b_per_w

        pltpu.sync_copy(idx_hbm.at[pl.ds(base, b_per_w)], idx_v)

        def start_gather(c, rows, sem):
            pltpu.async_copy(table_hbm.at[idx_v.at[pl.ds(c * _C, _C)]], rows, sem)

        def wait_rows(rows, sem):
            pltpu.make_async_copy(out_hbm.at[pl.ds(base, _C), pl.ds(0, _D)], rows, sem).wait()

        def write_rows(c, rows):
            pltpu.sync_copy(rows, out_hbm.at[pl.ds(base + c * _C, _C), pl.ds(0, _D)])

        start_gather(0, rows0, sem0)

        @pl.loop(0, n_chunks, step=2)
        def _(t):
            start_gather(t + 1, rows1, sem1)
            wait_rows(rows0, sem0)
            write_rows(t, rows0)
            start_gather(jnp.minimum(t + 2, n_chunks - 2), rows0, sem0)
            wait_rows(rows1, sem1)
            write_rows(t + 1, rows1)

        wait_rows(rows0, sem0)

    table_lin = transpose_kernel(table_t)
    out_padded = gather_kernel(table_lin, idx)
    out = _lane_slice_tc(out_padded.reshape(n * 4 * _D), n)
    return out.reshape(b, s, _D)


# speedup vs baseline: 1.0400x; 1.0400x over previous
import jax
import jax.numpy as jnp
from jax import lax
from jax.experimental import pallas as pl
from jax.experimental.pallas import tpu as pltpu
from jax.experimental.pallas import tpu_sc as plsc

_D = 32
_NC = 2
_NS = 16
_NW = _NC * _NS
_C = 512
_TW = 800

_SLICE_ROWS = 8192


def _lane_slice_tc(flat_padded, n):
    def body(i_ref, o_ref):
        o_ref[...] = i_ref[...].reshape(_SLICE_ROWS, 4 * _D)[:, :_D]

    return pl.pallas_call(
        body,
        grid=(n // _SLICE_ROWS,),
        in_specs=[pl.BlockSpec((_SLICE_ROWS * 4 * _D,), lambda i: (i,))],
        out_specs=pl.BlockSpec((_SLICE_ROWS, _D), lambda i: (i, 0)),
        out_shape=jax.ShapeDtypeStruct((n, _D), flat_padded.dtype),
        compiler_params=pltpu.CompilerParams(
            dimension_semantics=("parallel",)),
    )(flat_padded)


def kernel(x, table):
    b, s = x.shape
    n = b * s
    v = table.shape[0]
    idx = x.reshape(n)
    b_per_w = n // _NW
    n_chunks = b_per_w // _C
    v_per_w = v // _NW
    mesh = plsc.VectorSubcoreMesh(core_axis_name="c", subcore_axis_name="s")

    table_t = table.T

    @pl.kernel(
        out_type=jax.ShapeDtypeStruct((v, _D), table.dtype),
        mesh=mesh,
        compiler_params=pltpu.CompilerParams(
            use_tc_tiling_on_sc=False, needs_layout_passes=False),
        scratch_types=[
            pltpu.VMEM((_D, _TW), jnp.float32),
            pltpu.VMEM((_TW, _D), jnp.float32),
        ],
    )
    def transpose_kernel(tab_t_hbm, out_hbm, slab_v, outb_v):
        wid = lax.axis_index("s") * _NC + lax.axis_index("c")
        n_slabs = v // _TW
        n_rounds = (n_slabs + _NW - 1) // _NW
        iota16 = lax.iota(jnp.int32, 16)
        e_full = [jnp.full((16,), e, jnp.int32) for e in range(_D)]

        @pl.loop(0, n_rounds)
        def _(t):
            sid = wid + t * _NW

            @pl.when(sid < n_slabs)
            def _():
                c0 = sid * _TW
                pltpu.sync_copy(tab_t_hbm.at[:, pl.ds(c0, _TW)], slab_v)

                @pl.loop(0, _TW, step=16)
                def _(j):
                    jv = j + iota16
                    for e in range(_D):
                        plsc.store_scatter(
                            outb_v, [jv, e_full[e]], slab_v[e, pl.ds(j, 16)])

                pltpu.sync_copy(outb_v, out_hbm.at[pl.ds(c0, _TW)])

    @pl.kernel(
        out_type=jax.ShapeDtypeStruct((n, 4 * _D), table.dtype),
        mesh=mesh,
        compiler_params=pltpu.CompilerParams(use_tc_tiling_on_sc=False),
        scratch_types=[
            pltpu.VMEM((b_per_w,), jnp.int32),
            pltpu.VMEM((_C, _D), jnp.float32),
            pltpu.VMEM((_C, _D), jnp.float32),
            pltpu.SemaphoreType.DMA,
            pltpu.SemaphoreType.DMA,
        ],
    )
    def gather_kernel(table_hbm, idx_hbm, out_hbm, idx_v, rows0, rows1, sem0, sem1):
        wid = lax.axis_index("s") * _NC + lax.axis_index("c")
        base = wid * b_per_w

        pltpu.sync_copy(idx_hbm.at[pl.ds(base, b_per_w)], idx_v)

        def start_gather(c, rows, sem):
            pltpu.async_copy(table_hbm.at[idx_v.at[pl.ds(c * _C, _C)]], rows, sem)

        def wait_rows(rows, sem):
            pltpu.make_async_copy(out_hbm.at[pl.ds(base, _C), pl.ds(0, _D)], rows, sem).wait()

        def write_rows(c, rows):
            pltpu.sync_copy(rows, out_hbm.at[pl.ds(base + c * _C, _C), pl.ds(0, _D)])

        start_gather(0, rows0, sem0)

        @pl.loop(0, n_chunks, step=2)
        def _(t):
            start_gather(t + 1, rows1, sem1)
            wait_rows(rows0, sem0)
            write_rows(t, rows0)
            start_gather(jnp.minimum(t + 2, n_chunks - 2), rows0, sem0)
            wait_rows(rows1, sem1)
            write_rows(t + 1, rows1)

        wait_rows(rows0, sem0)

    table_lin = transpose_kernel(table_t)
    out_padded = gather_kernel(table_lin, idx)
    out = _lane_slice_tc(out_padded.reshape(n * 4 * _D), n)
    return out.reshape(b, s, _D)


# speedup vs baseline: 3.9423x; 3.7906x over previous
import jax
import jax.numpy as jnp
from jax import lax
from jax.experimental import pallas as pl
from jax.experimental.pallas import tpu as pltpu
from jax.experimental.pallas import tpu_sc as plsc

_D = 32
_NC = 2
_NS = 16
_NW = _NC * _NS
_C = 512
_TW = 800

_SLICE_ROWS = 8192


def _lane_slice_tc(flat_padded, n):
    def body(i_ref, o_ref):
        o_ref[...] = i_ref[...].reshape(_SLICE_ROWS, 4 * _D)[:, :_D]

    return pl.pallas_call(
        body,
        grid=(n // _SLICE_ROWS,),
        in_specs=[pl.BlockSpec((_SLICE_ROWS * 4 * _D,), lambda i: (i,))],
        out_specs=pl.BlockSpec((_SLICE_ROWS, _D), lambda i: (i, 0)),
        out_shape=jax.ShapeDtypeStruct((n, _D), flat_padded.dtype),
        compiler_params=pltpu.CompilerParams(
            dimension_semantics=("parallel",)),
    )(flat_padded)


def kernel(x, table):
    b, s = x.shape
    n = b * s
    v = table.shape[0]
    idx = x.reshape(n)
    b_per_w = n // _NW
    n_chunks = b_per_w // _C
    v_per_w = v // _NW
    mesh = plsc.VectorSubcoreMesh(core_axis_name="c", subcore_axis_name="s")

    table_t = table.T

    @pl.kernel(
        out_type=jax.ShapeDtypeStruct((v, _D), table.dtype),
        mesh=mesh,
        compiler_params=pltpu.CompilerParams(
            use_tc_tiling_on_sc=False, needs_layout_passes=False),
        scratch_types=[
            pltpu.VMEM((_D, _TW), jnp.float32),
            pltpu.VMEM((_TW, _D), jnp.float32),
        ],
    )
    def transpose_kernel(tab_t_hbm, out_hbm, slab_v, outb_v):
        wid = lax.axis_index("s") * _NC + lax.axis_index("c")
        n_slabs = v // _TW
        n_rounds = (n_slabs + _NW - 1) // _NW
        iota16 = lax.iota(jnp.int32, 16)
        e_full = [jnp.full((16,), e, jnp.int32) for e in range(_D)]

        @pl.loop(0, n_rounds)
        def _(t):
            sid = wid + t * _NW

            @pl.when(sid < n_slabs)
            def _():
                c0 = sid * _TW
                pltpu.sync_copy(tab_t_hbm.at[:, pl.ds(c0, _TW)], slab_v)

                @pl.loop(0, _TW, step=16)
                def _(j):
                    jv = j + iota16
                    for e in range(_D):
                        plsc.store_scatter(
                            outb_v, [jv, e_full[e]], slab_v[e, pl.ds(j, 16)])

                pltpu.sync_copy(outb_v, out_hbm.at[pl.ds(c0, _TW)])

    @pl.kernel(
        out_type=jax.ShapeDtypeStruct((n, 4 * _D), table.dtype),
        mesh=mesh,
        compiler_params=pltpu.CompilerParams(use_tc_tiling_on_sc=False),
        scratch_types=[
            pltpu.VMEM((b_per_w,), jnp.int32),
            pltpu.VMEM((_C, _D), jnp.float32),
            pltpu.VMEM((_C, _D), jnp.float32),
            pltpu.SemaphoreType.DMA,
            pltpu.SemaphoreType.DMA,
        ],
    )
    def gather_kernel(table_hbm, idx_hbm, out_hbm, idx_v, rows0, rows1, sem0, sem1):
        wid = lax.axis_index("s") * _NC + lax.axis_index("c")
        base = wid * b_per_w

        pltpu.sync_copy(idx_hbm.at[pl.ds(base, b_per_w)], idx_v)

        def start_gather(c, rows, sem):
            pltpu.async_copy(table_hbm.at[idx_v.at[pl.ds(c * _C, _C)]], rows, sem)

        def wait_rows(rows, sem):
            pltpu.make_async_copy(out_hbm.at[pl.ds(base, _C), pl.ds(0, _D)], rows, sem).wait()

        def write_rows(c, rows):
            pltpu.sync_copy(rows, out_hbm.at[pl.ds(base + c * _C, _C), pl.ds(0, _D)])

        start_gather(0, rows0, sem0)

        @pl.loop(0, n_chunks, step=2)
        def _(t):
            start_gather(t + 1, rows1, sem1)
            wait_rows(rows0, sem0)
            write_rows(t, rows0)
            start_gather(jnp.minimum(t + 2, n_chunks - 2), rows0, sem0)
            wait_rows(rows1, sem1)
            write_rows(t + 1, rows1)

        wait_rows(rows0, sem0)

    del transpose_kernel, table_t
    out_padded = gather_kernel(table, idx)
    out = _lane_slice_tc(out_padded.reshape(n * 4 * _D), n)
    return out.reshape(b, s, _D)


# speedup vs baseline: 5.3906x; 1.3674x over previous
import jax
import jax.numpy as jnp
from jax import lax
from jax.experimental import pallas as pl
from jax.experimental.pallas import tpu as pltpu
from jax.experimental.pallas import tpu_sc as plsc

_D = 32
_NC = 2
_NS = 16
_NW = _NC * _NS
_C = 512
_TW = 800

_SLICE_ROWS = 8192


def _lane_slice_tc(flat_padded, n):
    def body(i_ref, o_ref):
        o_ref[...] = i_ref[...].reshape(_SLICE_ROWS, 4 * _D)[:, :_D]

    return pl.pallas_call(
        body,
        grid=(n // _SLICE_ROWS,),
        in_specs=[pl.BlockSpec((_SLICE_ROWS * 4 * _D,), lambda i: (i,))],
        out_specs=pl.BlockSpec((_SLICE_ROWS, _D), lambda i: (i, 0)),
        out_shape=jax.ShapeDtypeStruct((n, _D), flat_padded.dtype),
        compiler_params=pltpu.CompilerParams(
            dimension_semantics=("parallel",)),
    )(flat_padded)


def kernel(x, table):
    b, s = x.shape
    n = b * s
    v = table.shape[0]
    idx = x.reshape(n)
    b_per_w = n // _NW
    n_chunks = b_per_w // _C
    v_per_w = v // _NW
    mesh = plsc.VectorSubcoreMesh(core_axis_name="c", subcore_axis_name="s")

    table_t = table.T

    @pl.kernel(
        out_type=jax.ShapeDtypeStruct((v, _D), table.dtype),
        mesh=mesh,
        compiler_params=pltpu.CompilerParams(
            use_tc_tiling_on_sc=False, needs_layout_passes=False),
        scratch_types=[
            pltpu.VMEM((_D, _TW), jnp.float32),
            pltpu.VMEM((_TW, _D), jnp.float32),
        ],
    )
    def transpose_kernel(tab_t_hbm, out_hbm, slab_v, outb_v):
        wid = lax.axis_index("s") * _NC + lax.axis_index("c")
        n_slabs = v // _TW
        n_rounds = (n_slabs + _NW - 1) // _NW
        iota16 = lax.iota(jnp.int32, 16)
        e_full = [jnp.full((16,), e, jnp.int32) for e in range(_D)]

        @pl.loop(0, n_rounds)
        def _(t):
            sid = wid + t * _NW

            @pl.when(sid < n_slabs)
            def _():
                c0 = sid * _TW
                pltpu.sync_copy(tab_t_hbm.at[:, pl.ds(c0, _TW)], slab_v)

                @pl.loop(0, _TW, step=16)
                def _(j):
                    jv = j + iota16
                    for e in range(_D):
                        plsc.store_scatter(
                            outb_v, [jv, e_full[e]], slab_v[e, pl.ds(j, 16)])

                pltpu.sync_copy(outb_v, out_hbm.at[pl.ds(c0, _TW)])

    @pl.kernel(
        out_type=jax.ShapeDtypeStruct((n, 4 * _D), table.dtype),
        mesh=mesh,
        compiler_params=pltpu.CompilerParams(use_tc_tiling_on_sc=False),
        scratch_types=[
            pltpu.VMEM((b_per_w,), jnp.int32),
            pltpu.VMEM((_C, _D), jnp.float32),
            pltpu.VMEM((_C, _D), jnp.float32),
            pltpu.SemaphoreType.DMA,
            pltpu.SemaphoreType.DMA,
        ],
    )
    def gather_kernel(table_hbm, idx_hbm, out_hbm, idx_v, rows0, rows1, sem0, sem1):
        wid = lax.axis_index("s") * _NC + lax.axis_index("c")
        base = wid * b_per_w

        pltpu.sync_copy(idx_hbm.at[pl.ds(base, b_per_w)], idx_v)

        def start_gather(c, rows, sem):
            pltpu.async_copy(table_hbm.at[idx_v.at[pl.ds(c * _C, _C)]], rows, sem)

        def wait_rows(rows, sem):
            pltpu.make_async_copy(out_hbm.at[pl.ds(base, _C), pl.ds(0, _D)], rows, sem).wait()

        def write_rows(c, rows):
            pltpu.sync_copy(rows, out_hbm.at[pl.ds(base + c * _C, _C), pl.ds(0, _D)])

        start_gather(0, rows0, sem0)

        @pl.loop(0, n_chunks, step=2)
        def _(t):
            start_gather(t + 1, rows1, sem1)
            wait_rows(rows0, sem0)
            write_rows(t, rows0)
            start_gather(jnp.minimum(t + 2, n_chunks - 2), rows0, sem0)
            wait_rows(rows1, sem1)
            write_rows(t + 1, rows1)

        wait_rows(rows0, sem0)

    del transpose_kernel, table_t
    out_padded = gather_kernel(table, idx)
    return out_padded[:, :_D].reshape(b, s, _D)
